# baseline (device time: 342887 ns/iter reference)
import jax
import jax.numpy as jnp
from jax import lax
from jax.experimental import pallas as pl
from jax.experimental.pallas import tpu as pltpu

N_DEV = 8
NS = 4


def kernel(x, w_mat):
    m, _ = x.shape
    _, n = w_mat.shape
    mc = m // N_DEV
    nh = n // 2
    ns = nh // NS

    def body(x_ref, w_ref, out_ref, comm_ref, send_sems, recv_sems,
             credit_sems):
        my = lax.axis_index("i")
        right = lax.rem(my + 1, N_DEV)
        left = lax.rem(my + N_DEV - 1, N_DEV)

        barrier = pltpu.get_barrier_semaphore()
        for nbr in (left, right):
            pl.semaphore_signal(
                barrier, inc=1, device_id=(nbr,),
                device_id_type=pl.DeviceIdType.MESH,
            )
        pl.semaphore_wait(barrier, 2)

        rings = [(d, k) for k in range(NS) for d in (0, 1)]

        def dst_of(d):
            return right if d == 0 else left

        def src_of(d):
            return left if d == 0 else right

        def strip_dot(c, d, k):
            xc = x_ref[pl.ds(c * mc, mc), :].astype(jnp.bfloat16)
            wc = w_ref[:, pl.ds(d * nh + k * ns, ns)].astype(jnp.bfloat16)
            return jnp.dot(xc, wc, preferred_element_type=jnp.float32)

        def rdma(d, k, h):
            return pltpu.make_async_remote_copy(
                src_ref=comm_ref.at[d, k, h % 2],
                dst_ref=comm_ref.at[d, k, (h + 1) % 2],
                send_sem=send_sems.at[d, k, h % 2],
                recv_sem=recv_sems.at[d, k, (h + 1) % 2],
                device_id=(dst_of(d),),
                device_id_type=pl.DeviceIdType.MESH,
            )

        for d, k in rings:
            c = left if d == 0 else right
            comm_ref[d, k, 0] = strip_dot(c, d, k).astype(jnp.bfloat16)
            rdma(d, k, 0).start()

        for h in range(N_DEV - 1):
            last = h == N_DEV - 2
            for d, k in rings:
                slot_r = (h + 1) % 2
                desc = rdma(d, k, h)
                desc.wait_recv()
                desc.wait_send()
                if not last:
                    pl.semaphore_signal(
                        credit_sems.at[d, k], inc=1,
                        device_id=(src_of(d),),
                        device_id_type=pl.DeviceIdType.MESH,
                    )
                c = lax.rem(my + 2 * N_DEV - 2 - h, N_DEV) if d == 0 else (
                    lax.rem(my + 2 + h, N_DEV))
                seg = (comm_ref[d, k, slot_r].astype(jnp.float32)
                       + strip_dot(c, d, k))
                if last:
                    out_ref[:, pl.ds(d * nh + k * ns, ns)] = jnp.maximum(
                        seg, 0.0).astype(jnp.bfloat16)
                else:
                    comm_ref[d, k, slot_r] = seg.astype(jnp.bfloat16)
                    pl.semaphore_wait(credit_sems.at[d, k], 1)
                    rdma(d, k, h + 1).start()

    return pl.pallas_call(
        body,
        out_shape=jax.ShapeDtypeStruct((mc, n), jnp.bfloat16),
        in_specs=[
            pl.BlockSpec(memory_space=pltpu.VMEM),
            pl.BlockSpec(memory_space=pltpu.VMEM),
        ],
        out_specs=pl.BlockSpec(memory_space=pltpu.VMEM),
        scratch_shapes=[
            pltpu.VMEM((2, NS, 2, mc, ns), jnp.bfloat16),
            pltpu.SemaphoreType.DMA((2, NS, 2)),
            pltpu.SemaphoreType.DMA((2, NS, 2)),
            pltpu.SemaphoreType.REGULAR((2, NS)),
        ],
        compiler_params=pltpu.CompilerParams(
            collective_id=0,
            vmem_limit_bytes=58 * 1024 * 1024,
        ),
    )(x, w_mat)
